# Initial kernel scaffold; baseline (speedup 1.0000x reference)
import jax
import jax.numpy as jnp
from jax.experimental import pallas as pl


def kernel(x, edge_index, adj_values, W1, b1, W2, b2):
    raise NotImplementedError("write your pallas kernel here")


# speedup vs baseline: 6.5061x; 6.5061x over previous
import functools

import jax
import jax.numpy as jnp
from jax import lax
from jax.experimental import pallas as pl
from jax.experimental.pallas import tpu as pltpu
from jax.experimental.pallas import tpu_sc as plsc

N = 10000
E = 320000
D = 128

NC = 2
NS = 16
NW = NC * NS
EPW = E // NW
CHUNK = 80
SCHUNK = 25
SEDGE = SCHUNK * CHUNK
NSTAGE = EPW // SEDGE
RPW = 624
ZROWS = 48
TAIL = N - NS * RPW


def _spmm_sc(x, col, row4, val):
    mesh = plsc.VectorSubcoreMesh(core_axis_name="c", subcore_axis_name="s")

    @functools.partial(
        pl.kernel,
        out_type=jax.ShapeDtypeStruct((NC, N, D), jnp.float32),
        mesh=mesh,
        scratch_types=[
            pltpu.VMEM_SHARED((N, D), jnp.float32),
            pltpu.VMEM((SEDGE,), jnp.int32),
            pltpu.VMEM((SCHUNK, CHUNK), jnp.int32),
            pltpu.VMEM((SEDGE,), jnp.float32),
            pltpu.VMEM((CHUNK, D), jnp.float32),
            pltpu.VMEM((ZROWS, D), jnp.float32),
            pltpu.SemaphoreType.DMA,
        ],
    )
    def spmm(x_hbm, col_hbm, row_hbm, val_hbm, out_hbm,
             acc, col_v, row_v, val_v, gbuf, zbuf, sem):
        c = lax.axis_index("c")
        s = lax.axis_index("s")
        wid = s * NC + c

        @pl.loop(0, ZROWS)
        def _zero(i):
            for t in range(D // 16):
                zbuf.at[i, pl.ds(t * 16, 16)][...] = jnp.zeros(
                    (16,), jnp.float32)

        for j in range(RPW // ZROWS):
            pltpu.sync_copy(
                zbuf, acc.at[pl.ds(s * RPW + j * ZROWS, ZROWS)])

        @pl.when(s == 0)
        def _zero_tail():
            pltpu.sync_copy(zbuf.at[pl.ds(0, TAIL)],
                            acc.at[pl.ds(NS * RPW, TAIL)])

        plsc.subcore_barrier()

        @pl.loop(0, NSTAGE)
        def _stage(st):
            base_e = wid * EPW + st * SEDGE
            pltpu.sync_copy(col_hbm.at[pl.ds(base_e, SEDGE)], col_v)
            pltpu.sync_copy(row_hbm.at[wid, st], row_v)
            pltpu.sync_copy(val_hbm.at[pl.ds(base_e, SEDGE)], val_v)

            @pl.loop(0, SCHUNK)
            def _chunk(k):
                cidx = col_v.at[pl.ds(k * CHUNK, CHUNK)]
                pltpu.async_copy(x_hbm.at[cidx], gbuf, sem).wait()

                @pl.loop(0, CHUNK, step=16)
                def _scale(e0):
                    vals16 = val_v[pl.ds(k * CHUNK + e0, 16)]
                    for j in range(16):
                        vv = lax.broadcast(vals16[j], (16,))
                        for t in range(D // 16):
                            sl = (e0 + j, pl.ds(t * 16, 16))
                            gbuf.at[sl][...] = gbuf.at[sl][...] * vv

                pltpu.sync_copy(gbuf, acc.at[row_v.at[k]], add=True)

        plsc.subcore_barrier()
        pltpu.sync_copy(acc.at[pl.ds(s * RPW, RPW)],
                        out_hbm.at[c, pl.ds(s * RPW, RPW)])

        @pl.when(s == 0)
        def _write_tail():
            pltpu.sync_copy(acc.at[pl.ds(NS * RPW, TAIL)],
                            out_hbm.at[c, pl.ds(NS * RPW, TAIL)])

    return spmm(x, col, row4, val)


def _linear_tc(parts, W, b, relu):
    def body(p_ref, w_ref, b_ref, o_ref):
        h = p_ref[0] + p_ref[1]
        y = jnp.dot(h, w_ref[...], preferred_element_type=jnp.float32)
        y = y + b_ref[...]
        if relu:
            y = jnp.maximum(y, 0.0)
        o_ref[...] = y

    return pl.pallas_call(
        body,
        out_shape=jax.ShapeDtypeStruct((N, D), jnp.float32),
    )(parts, W, b.reshape(1, D))


def kernel(x, edge_index, adj_values, W1, b1, W2, b2):
    row = edge_index[0]
    col = edge_index[1]
    row4 = row.reshape(NW, NSTAGE, SCHUNK, CHUNK)

    p1 = _spmm_sc(x, col, row4, adj_values)
    h = _linear_tc(p1, W1, b1, relu=True)
    p2 = _spmm_sc(h, col, row4, adj_values)
    out = _linear_tc(p2, W2, b2, relu=False)
    return out


# speedup vs baseline: 9.9666x; 1.5319x over previous
import functools

import jax
import jax.numpy as jnp
from jax import lax
from jax.experimental import pallas as pl
from jax.experimental.pallas import tpu as pltpu
from jax.experimental.pallas import tpu_sc as plsc

N = 10000
E = 320000
D = 128

NC = 2
NS = 16
NW = NC * NS
EPW = E // NW
CHUNK = 80
SCHUNK = 25
SEDGE = SCHUNK * CHUNK
NSTAGE = EPW // SEDGE
RPW = 624
ZROWS = 48
TAIL = N - NS * RPW


def _spmm_sc(x, col, row4, val):
    mesh = plsc.VectorSubcoreMesh(core_axis_name="c", subcore_axis_name="s")

    @functools.partial(
        pl.kernel,
        out_type=jax.ShapeDtypeStruct((NC, N, D), jnp.float32),
        mesh=mesh,
        scratch_types=[
            pltpu.VMEM_SHARED((N, D), jnp.float32),
            pltpu.VMEM((SEDGE,), jnp.int32),
            pltpu.VMEM((SEDGE,), jnp.int32),
            pltpu.VMEM((SCHUNK, CHUNK), jnp.int32),
            pltpu.VMEM((SCHUNK, CHUNK), jnp.int32),
            pltpu.VMEM((SEDGE,), jnp.float32),
            pltpu.VMEM((SEDGE,), jnp.float32),
            pltpu.VMEM((CHUNK, D), jnp.float32),
            pltpu.VMEM((CHUNK, D), jnp.float32),
            pltpu.VMEM((ZROWS, D), jnp.float32),
            pltpu.SemaphoreType.DMA,
            pltpu.SemaphoreType.DMA,
            pltpu.SemaphoreType.DMA,
            pltpu.SemaphoreType.DMA,
            pltpu.SemaphoreType.DMA,
        ],
    )
    def spmm(x_hbm, col_hbm, row_hbm, val_hbm, out_hbm,
             acc, col_a, col_b, row_a, row_b, val_a, val_b, g0, g1, zbuf,
             sg0, sg1, ss0, ss1, sst):
        c = lax.axis_index("c")
        s = lax.axis_index("s")
        wid = s * NC + c
        cols = (col_a, col_b)
        rows = (row_a, row_b)
        vals = (val_a, val_b)

        def stage_copies(st_idx, p):
            base_e = wid * EPW + st_idx * SEDGE
            return (
                pltpu.make_async_copy(
                    col_hbm.at[pl.ds(base_e, SEDGE)], cols[p], sst),
                pltpu.make_async_copy(
                    row_hbm.at[wid, st_idx], rows[p], sst),
                pltpu.make_async_copy(
                    val_hbm.at[pl.ds(base_e, SEDGE)], vals[p], sst),
            )

        for cp in stage_copies(0, 0):
            cp.start()

        @pl.loop(0, ZROWS)
        def _zero(i):
            for t in range(D // 16):
                zbuf.at[i, pl.ds(t * 16, 16)][...] = jnp.zeros(
                    (16,), jnp.float32)

        for j in range(RPW // ZROWS):
            pltpu.sync_copy(
                zbuf, acc.at[pl.ds(s * RPW + j * ZROWS, ZROWS)])

        @pl.when(s == 0)
        def _zero_tail():
            pltpu.sync_copy(zbuf.at[pl.ds(0, TAIL)],
                            acc.at[pl.ds(NS * RPW, TAIL)])

        plsc.subcore_barrier()

        for st in range(NSTAGE):
            p = st % 2
            for cp in stage_copies(st, p):
                cp.wait()
            if st + 1 < NSTAGE:
                for cp in stage_copies(st + 1, 1 - p):
                    cp.start()

            def gather(k, gb, sem):
                cidx = cols[p].at[pl.ds(k * CHUNK, CHUNK)]
                return pltpu.make_async_copy(x_hbm.at[cidx], gb, sem)

            def scatter(k, gb, sem):
                return pltpu.make_async_copy(gb, acc.at[rows[p].at[k]], sem)

            def scale(k, gb):
                @pl.loop(0, CHUNK, step=16)
                def _scale(e0):
                    vals16 = vals[p][pl.ds(k * CHUNK + e0, 16)]
                    for j in range(16):
                        vv = lax.broadcast(vals16[j], (16,))
                        for t in range(D // 16):
                            sl = (e0 + j, pl.ds(t * 16, 16))
                            gb.at[sl][...] = gb.at[sl][...] * vv

            gather(0, g0, sg0).start()
            gather(1, g1, sg1).start()

            @pl.loop(0, SCHUNK // 2)
            def _pair(k2):
                c0 = 2 * k2
                gather(c0, g0, sg0).wait()
                scale(c0, g0)
                scatter(c0, g0, ss0).start(add=True)
                gather(c0 + 1, g1, sg1).wait()
                scale(c0 + 1, g1)
                scatter(c0 + 1, g1, ss1).start(add=True)

                @pl.when(k2 < SCHUNK // 2 - 1)
                def _more():
                    scatter(c0, g0, ss0).wait()
                    gather(c0 + 2, g0, sg0).start()
                    scatter(c0 + 1, g1, ss1).wait()
                    gather(c0 + 3, g1, sg1).start()

                @pl.when(k2 == SCHUNK // 2 - 1)
                def _last():
                    scatter(c0, g0, ss0).wait()
                    gather(SCHUNK - 1, g0, sg0).start()
                    scatter(c0 + 1, g1, ss1).wait()

            gather(SCHUNK - 1, g0, sg0).wait()
            scale(SCHUNK - 1, g0)
            scatter(SCHUNK - 1, g0, ss0).start(add=True)
            scatter(SCHUNK - 1, g0, ss0).wait()

        plsc.subcore_barrier()
        pltpu.sync_copy(acc.at[pl.ds(s * RPW, RPW)],
                        out_hbm.at[c, pl.ds(s * RPW, RPW)])

        @pl.when(s == 0)
        def _write_tail():
            pltpu.sync_copy(acc.at[pl.ds(NS * RPW, TAIL)],
                            out_hbm.at[c, pl.ds(NS * RPW, TAIL)])

    return spmm(x, col, row4, val)


def _linear_tc(parts, W, b, relu):
    def body(p_ref, w_ref, b_ref, o_ref):
        h = p_ref[0] + p_ref[1]
        y = jnp.dot(h, w_ref[...], preferred_element_type=jnp.float32)
        y = y + b_ref[...]
        if relu:
            y = jnp.maximum(y, 0.0)
        o_ref[...] = y

    return pl.pallas_call(
        body,
        out_shape=jax.ShapeDtypeStruct((N, D), jnp.float32),
    )(parts, W, b.reshape(1, D))


def kernel(x, edge_index, adj_values, W1, b1, W2, b2):
    row = edge_index[0]
    col = edge_index[1]
    row4 = row.reshape(NW, NSTAGE, SCHUNK, CHUNK)

    p1 = _spmm_sc(x, col, row4, adj_values)
    h = _linear_tc(p1, W1, b1, relu=True)
    p2 = _spmm_sc(h, col, row4, adj_values)
    out = _linear_tc(p2, W2, b2, relu=False)
    return out


# speedup vs baseline: 11.9657x; 1.2006x over previous
import functools

import jax
import jax.numpy as jnp
from jax import lax
from jax.experimental import pallas as pl
from jax.experimental.pallas import tpu as pltpu
from jax.experimental.pallas import tpu_sc as plsc

N = 10000
E = 320000
D = 128

NC = 2
NS = 16
NW = NC * NS
EPW = E // NW
CHUNK = 80
SCHUNK = 25
SEDGE = SCHUNK * CHUNK
NSTAGE = EPW // SEDGE
RPW = 624
ZROWS = 16
TAIL = N - NS * RPW


def _spmm_sc(x, col, row4, val):
    mesh = plsc.VectorSubcoreMesh(core_axis_name="c", subcore_axis_name="s")

    @functools.partial(
        pl.kernel,
        out_type=jax.ShapeDtypeStruct((NC, N, D), jnp.float32),
        mesh=mesh,
        scratch_types=[
            pltpu.VMEM_SHARED((N, D), jnp.float32),
            pltpu.VMEM((SEDGE,), jnp.int32),
            pltpu.VMEM((SEDGE,), jnp.int32),
            pltpu.VMEM((SCHUNK, CHUNK), jnp.int32),
            pltpu.VMEM((SCHUNK, CHUNK), jnp.int32),
            pltpu.VMEM((SEDGE,), jnp.float32),
            pltpu.VMEM((SEDGE,), jnp.float32),
            pltpu.VMEM((CHUNK, D), jnp.float32),
            pltpu.VMEM((CHUNK, D), jnp.float32),
            pltpu.VMEM((CHUNK, D), jnp.float32),
            pltpu.VMEM((ZROWS, D), jnp.float32),
            pltpu.SemaphoreType.DMA,
            pltpu.SemaphoreType.DMA,
            pltpu.SemaphoreType.DMA,
            pltpu.SemaphoreType.DMA,
            pltpu.SemaphoreType.DMA,
            pltpu.SemaphoreType.DMA,
            pltpu.SemaphoreType.DMA,
        ],
    )
    def spmm(x_hbm, col_hbm, row_hbm, val_hbm, out_hbm,
             acc, col_a, col_b, row_a, row_b, val_a, val_b, g0, g1, g2, zbuf,
             sg0, sg1, sg2, ss0, ss1, ss2, sst):
        c = lax.axis_index("c")
        s = lax.axis_index("s")
        wid = s * NC + c
        cols = (col_a, col_b)
        rows = (row_a, row_b)
        vals = (val_a, val_b)

        def stage_copies(st_idx, p):
            base_e = wid * EPW + st_idx * SEDGE
            return (
                pltpu.make_async_copy(
                    col_hbm.at[pl.ds(base_e, SEDGE)], cols[p], sst),
                pltpu.make_async_copy(
                    row_hbm.at[wid, st_idx], rows[p], sst),
                pltpu.make_async_copy(
                    val_hbm.at[pl.ds(base_e, SEDGE)], vals[p], sst),
            )

        for cp in stage_copies(0, 0):
            cp.start()

        @pl.loop(0, ZROWS)
        def _zero(i):
            for t in range(D // 16):
                zbuf.at[i, pl.ds(t * 16, 16)][...] = jnp.zeros(
                    (16,), jnp.float32)

        def zero_copies():
            return [pltpu.make_async_copy(
                        zbuf, acc.at[pl.ds(s * RPW + j * ZROWS, ZROWS)], ss0)
                    for j in range(RPW // ZROWS)]

        for cp in zero_copies():
            cp.start()

        @pl.when(s == 0)
        def _zero_tail():
            pltpu.sync_copy(zbuf.at[pl.ds(0, TAIL)],
                            acc.at[pl.ds(NS * RPW, TAIL)])

        for cp in zero_copies():
            cp.wait()
        plsc.subcore_barrier()

        for st in range(NSTAGE):
            p = st % 2
            for cp in stage_copies(st, p):
                cp.wait()
            if st + 1 < NSTAGE:
                for cp in stage_copies(st + 1, 1 - p):
                    cp.start()

            gbufs = (g0, g1, g2)
            sgs = (sg0, sg1, sg2)
            sss = (ss0, ss1, ss2)

            def gather(k, r):
                cidx = cols[p].at[pl.ds(k * CHUNK, CHUNK)]
                return pltpu.make_async_copy(x_hbm.at[cidx], gbufs[r],
                                             sgs[r])

            def scatter(k, r):
                return pltpu.make_async_copy(gbufs[r], acc.at[rows[p].at[k]],
                                             sss[r])

            def scale(k, r):
                gb = gbufs[r]

                @pl.loop(0, CHUNK, step=16)
                def _scale(e0):
                    vals16 = vals[p][pl.ds(k * CHUNK + e0, 16)]
                    for j in range(16):
                        vv = lax.broadcast(vals16[j], (16,))
                        for t in range(D // 16):
                            sl = (e0 + j, pl.ds(t * 16, 16))
                            gb.at[sl][...] = gb.at[sl][...] * vv

            gather(0, 0).start()
            gather(1, 1).start()

            @pl.loop(0, SCHUNK // 3)
            def _triple(t):
                for r in range(3):
                    cc = 3 * t + r
                    if r == 0:
                        @pl.when(t > 0)
                        def _free():
                            scatter(cc - 1, 2).wait()

                        gather(cc + 2, 2).start()
                    elif r == 1:
                        scatter(cc - 1, 0).wait()
                        gather(cc + 2, 0).start()
                    else:
                        scatter(cc - 1, 1).wait()

                        @pl.when(t < SCHUNK // 3 - 1)
                        def _ahead():
                            gather(cc + 2, 1).start()

                    gather(cc, r).wait()
                    scale(cc, r)
                    scatter(cc, r).start(add=True)

            last = SCHUNK - 1
            scatter(last - 1, 2).wait()
            gather(last, 0).wait()
            scale(last, 0)
            scatter(last, 0).start(add=True)
            scatter(last, 0).wait()

        plsc.subcore_barrier()
        pltpu.sync_copy(acc.at[pl.ds(s * RPW, RPW)],
                        out_hbm.at[c, pl.ds(s * RPW, RPW)])

        @pl.when(s == 0)
        def _write_tail():
            pltpu.sync_copy(acc.at[pl.ds(NS * RPW, TAIL)],
                            out_hbm.at[c, pl.ds(NS * RPW, TAIL)])

    return spmm(x, col, row4, val)


def _linear_tc(parts, W, b, relu):
    def body(p_ref, w_ref, b_ref, o_ref):
        h = p_ref[0] + p_ref[1]
        y = jnp.dot(h, w_ref[...], preferred_element_type=jnp.float32)
        y = y + b_ref[...]
        if relu:
            y = jnp.maximum(y, 0.0)
        o_ref[...] = y

    return pl.pallas_call(
        body,
        out_shape=jax.ShapeDtypeStruct((N, D), jnp.float32),
    )(parts, W, b.reshape(1, D))


def kernel(x, edge_index, adj_values, W1, b1, W2, b2):
    row = edge_index[0]
    col = edge_index[1]
    row4 = row.reshape(NW, NSTAGE, SCHUNK, CHUNK)

    p1 = _spmm_sc(x, col, row4, adj_values)
    h = _linear_tc(p1, W1, b1, relu=True)
    p2 = _spmm_sc(h, col, row4, adj_values)
    out = _linear_tc(p2, W2, b2, relu=False)
    return out
